# Initial kernel scaffold; baseline (speedup 1.0000x reference)
import jax
import jax.numpy as jnp
from jax.experimental import pallas as pl


def kernel(inSen, adj, emb, W, a, Wc, bc):
    raise NotImplementedError("write your pallas kernel here")


# speedup vs baseline: 1.4421x; 1.4421x over previous
import functools

import jax
import jax.numpy as jnp
from jax import lax
from jax.experimental import pallas as pl
from jax.experimental.pallas import tpu as pltpu
from jax.experimental.pallas import tpu_sc as plsc

N = 4096
EDIM = 64
WFEAT = 64
LABELS = 2
SLOPE = 0.01
BLK = 512
NEG = -9e15


def _sc_gather(emb, idx):
    info = plsc.get_sparse_core_info()
    nc, ns = info.num_cores, info.num_subcores
    nw = nc * ns
    b = idx.shape[0]
    d = emb.shape[1]
    b_per_w = b // nw
    mesh = plsc.VectorSubcoreMesh(core_axis_name="c", subcore_axis_name="s")

    @functools.partial(
        pl.kernel,
        mesh=mesh,
        compiler_params=pltpu.CompilerParams(use_tc_tiling_on_sc=False),
        out_type=jax.ShapeDtypeStruct((b, d), jnp.float32),
        scratch_types=[
            pltpu.VMEM((b_per_w,), jnp.int32),
            pltpu.VMEM((b_per_w, d), jnp.float32),
            pltpu.SemaphoreType.DMA,
        ],
    )
    def k(table_hbm, idx_hbm, out_hbm, idx_v, rows_v, sem):
        wid = lax.axis_index("s") * nc + lax.axis_index("c")
        base = wid * b_per_w
        pltpu.sync_copy(idx_hbm.at[pl.ds(base, b_per_w)], idx_v)
        pltpu.async_copy(table_hbm.at[idx_v], rows_v, sem).wait()
        pltpu.sync_copy(rows_v, out_hbm.at[pl.ds(base, b_per_w)])

    return k(emb, idx)


def _gat_body(words_ref, w_ref, a1_ref, a2t_ref, wc_ref, bc_ref, adj_ref,
              att_ref, sent_ref, pool_ref, label_ref,
              wh_ref, f2t_ref, acc_ref):
    i = pl.program_id(0)
    nblk = pl.num_programs(0)

    @pl.when(i == 0)
    def _init():
        wh = jnp.dot(words_ref[...], w_ref[...],
                     preferred_element_type=jnp.float32)
        wh_ref[...] = wh
        f2t_ref[...] = lax.dot_general(
            a2t_ref[...], wh, (((1,), (1,)), ((), ())),
            preferred_element_type=jnp.float32)
        acc_ref[...] = jnp.zeros_like(acc_ref)

    wh_all = wh_ref[...]
    wh_blk = wh_ref[pl.ds(i * BLK, BLK), :]
    f1 = jnp.dot(wh_blk, a1_ref[...], preferred_element_type=jnp.float32)
    e = f1 + f2t_ref[...]
    e = jnp.where(e >= 0, e, SLOPE * e)
    e = jnp.where(adj_ref[...] > 0, e, NEG)
    m = jnp.max(e, axis=1, keepdims=True)
    p = jnp.exp(e - m)
    att = p / jnp.sum(p, axis=1, keepdims=True)
    att_ref[...] = att
    h = jnp.dot(att, wh_all, preferred_element_type=jnp.float32)
    sent = jnp.where(h > 0, h, jnp.exp(jnp.minimum(h, 0.0)) - 1.0)
    sent_ref[...] = sent
    acc_ref[...] += jnp.sum(sent, axis=0, keepdims=True)

    @pl.when(i == nblk - 1)
    def _fin():
        pool = acc_ref[...] * (1.0 / N)
        pool_ref[...] = pool
        logits = jnp.dot(pool, wc_ref[...],
                         preferred_element_type=jnp.float32) + bc_ref[...]
        mm = jnp.max(logits, axis=1, keepdims=True)
        pe = jnp.exp(logits - mm)
        label_ref[...] = pe / jnp.sum(pe, axis=1, keepdims=True)


def _gat(words, adj, W, a1, a2t, Wc, bc2):
    nblk = N // BLK
    out_shapes = (
        jax.ShapeDtypeStruct((N, N), jnp.float32),
        jax.ShapeDtypeStruct((N, WFEAT), jnp.float32),
        jax.ShapeDtypeStruct((1, WFEAT), jnp.float32),
        jax.ShapeDtypeStruct((1, LABELS), jnp.float32),
    )
    return pl.pallas_call(
        _gat_body,
        grid=(nblk,),
        in_specs=[
            pl.BlockSpec((N, EDIM), lambda i: (0, 0)),
            pl.BlockSpec((EDIM, WFEAT), lambda i: (0, 0)),
            pl.BlockSpec((WFEAT, 1), lambda i: (0, 0)),
            pl.BlockSpec((1, WFEAT), lambda i: (0, 0)),
            pl.BlockSpec((WFEAT, LABELS), lambda i: (0, 0)),
            pl.BlockSpec((1, LABELS), lambda i: (0, 0)),
            pl.BlockSpec((BLK, N), lambda i: (i, 0)),
        ],
        out_specs=(
            pl.BlockSpec((BLK, N), lambda i: (i, 0)),
            pl.BlockSpec((BLK, WFEAT), lambda i: (i, 0)),
            pl.BlockSpec((1, WFEAT), lambda i: (0, 0)),
            pl.BlockSpec((1, LABELS), lambda i: (0, 0)),
        ),
        out_shape=out_shapes,
        scratch_shapes=[
            pltpu.VMEM((N, WFEAT), jnp.float32),
            pltpu.VMEM((1, N), jnp.float32),
            pltpu.VMEM((1, WFEAT), jnp.float32),
        ],
    )(words, W, a1, a2t, Wc, bc2, adj)


def kernel(inSen, adj, emb, W, a, Wc, bc):
    idx = inSen.astype(jnp.int32)
    words = _sc_gather(emb, idx)
    a1 = a[:WFEAT, :]
    a2t = a[WFEAT:, :].reshape(1, WFEAT)
    bc2 = bc.reshape(1, LABELS)
    attention, sentence, pool, label = _gat(words, adj, W, a1, a2t, Wc, bc2)
    return (pool.reshape(WFEAT), attention, sentence, label.reshape(LABELS))


# speedup vs baseline: 1.7616x; 1.2215x over previous
import functools

import jax
import jax.numpy as jnp
from jax import lax
from jax.experimental import pallas as pl
from jax.experimental.pallas import tpu as pltpu
from jax.experimental.pallas import tpu_sc as plsc

N = 4096
EDIM = 64
WFEAT = 64
LABELS = 2
SLOPE = 0.01
BLK = 512
NEG = -9e15
FIRE = 16


def _sc_gather_rows(emb, idx):
    info = plsc.get_sparse_core_info()
    nc, ns = info.num_cores, info.num_subcores
    nw = nc * ns
    b = idx.shape[0]
    d = emb.shape[1]
    b_per_w = b // nw
    n_rounds = b_per_w // FIRE
    mesh = plsc.VectorSubcoreMesh(core_axis_name="c", subcore_axis_name="s")

    @functools.partial(
        pl.kernel,
        mesh=mesh,
        out_type=jax.ShapeDtypeStruct((b, d), jnp.float32),
        scratch_types=[
            pltpu.VMEM((b_per_w,), jnp.int32),
            pltpu.VMEM((b_per_w, d), jnp.float32),
            pltpu.SemaphoreType.DMA,
            pltpu.SemaphoreType.DMA,
        ],
    )
    def k(table_hbm, idx_hbm, out_hbm, idx_v, rows_v, sem, sem2):
        wid = lax.axis_index("s") * nc + lax.axis_index("c")
        base = wid * b_per_w
        pltpu.sync_copy(idx_hbm.at[pl.ds(base, b_per_w)], idx_v)

        def round_body(g, _):
            j0 = g * FIRE
            iv = idx_v[pl.ds(j0, FIRE)]
            cp = None
            for t in range(FIRE):
                r = iv[t]
                cp = pltpu.async_copy(
                    table_hbm.at[pl.ds(r, 1), :],
                    rows_v.at[pl.ds(j0 + t, 1), :], sem)
            for t in range(FIRE):
                cp.wait()
            return 0

        lax.fori_loop(0, n_rounds, round_body, 0)
        pltpu.async_copy(rows_v, out_hbm.at[pl.ds(base, b_per_w)], sem2).wait()

    return k(emb, idx)


def _gat_body(words_ref, w_ref, a1_ref, a2t_ref, wc_ref, bc_ref, adj_ref,
              att_ref, sent_ref, pool_ref, label_ref,
              wh_ref, f2t_ref, acc_ref):
    i = pl.program_id(0)
    nblk = pl.num_programs(0)

    @pl.when(i == 0)
    def _init():
        wh = jnp.dot(words_ref[...], w_ref[...],
                     preferred_element_type=jnp.float32)
        wh_ref[...] = wh
        f2t_ref[...] = lax.dot_general(
            a2t_ref[...], wh, (((1,), (1,)), ((), ())),
            preferred_element_type=jnp.float32)
        acc_ref[...] = jnp.zeros_like(acc_ref)

    wh_all = wh_ref[...]
    wh_blk = wh_ref[pl.ds(i * BLK, BLK), :]
    f1 = jnp.dot(wh_blk, a1_ref[...], preferred_element_type=jnp.float32)
    e = f1 + f2t_ref[...]
    e = jnp.where(e >= 0, e, SLOPE * e)
    e = jnp.where(adj_ref[...] > 0, e, NEG)
    m = jnp.max(e, axis=1, keepdims=True)
    p = jnp.exp(e - m)
    att = p / jnp.sum(p, axis=1, keepdims=True)
    att_ref[...] = att
    h = jnp.dot(att, wh_all, preferred_element_type=jnp.float32)
    sent = jnp.where(h > 0, h, jnp.exp(jnp.minimum(h, 0.0)) - 1.0)
    sent_ref[...] = sent
    acc_ref[...] += jnp.sum(sent, axis=0, keepdims=True)

    @pl.when(i == nblk - 1)
    def _fin():
        pool = acc_ref[...] * (1.0 / N)
        pool_ref[...] = pool
        logits = jnp.dot(pool, wc_ref[...],
                         preferred_element_type=jnp.float32) + bc_ref[...]
        mm = jnp.max(logits, axis=1, keepdims=True)
        pe = jnp.exp(logits - mm)
        label_ref[...] = pe / jnp.sum(pe, axis=1, keepdims=True)


def _gat(words, adj, W, a1, a2t, Wc, bc2):
    nblk = N // BLK
    out_shapes = (
        jax.ShapeDtypeStruct((N, N), jnp.float32),
        jax.ShapeDtypeStruct((N, WFEAT), jnp.float32),
        jax.ShapeDtypeStruct((1, WFEAT), jnp.float32),
        jax.ShapeDtypeStruct((1, LABELS), jnp.float32),
    )
    return pl.pallas_call(
        _gat_body,
        grid=(nblk,),
        in_specs=[
            pl.BlockSpec((N, EDIM), lambda i: (0, 0)),
            pl.BlockSpec((EDIM, WFEAT), lambda i: (0, 0)),
            pl.BlockSpec((WFEAT, 1), lambda i: (0, 0)),
            pl.BlockSpec((1, WFEAT), lambda i: (0, 0)),
            pl.BlockSpec((WFEAT, LABELS), lambda i: (0, 0)),
            pl.BlockSpec((1, LABELS), lambda i: (0, 0)),
            pl.BlockSpec((BLK, N), lambda i: (i, 0)),
        ],
        out_specs=(
            pl.BlockSpec((BLK, N), lambda i: (i, 0)),
            pl.BlockSpec((BLK, WFEAT), lambda i: (i, 0)),
            pl.BlockSpec((1, WFEAT), lambda i: (0, 0)),
            pl.BlockSpec((1, LABELS), lambda i: (0, 0)),
        ),
        out_shape=out_shapes,
        scratch_shapes=[
            pltpu.VMEM((N, WFEAT), jnp.float32),
            pltpu.VMEM((1, N), jnp.float32),
            pltpu.VMEM((1, WFEAT), jnp.float32),
        ],
    )(words, W, a1, a2t, Wc, bc2, adj)


def kernel(inSen, adj, emb, W, a, Wc, bc):
    idx = inSen.astype(jnp.int32)
    words = _sc_gather_rows(emb, idx)
    a1 = a[:WFEAT, :]
    a2t = a[WFEAT:, :].reshape(1, WFEAT)
    bc2 = bc.reshape(1, LABELS)
    attention, sentence, pool, label = _gat(words, adj, W, a1, a2t, Wc, bc2)
    return (pool.reshape(WFEAT), attention, sentence, label.reshape(LABELS))


# speedup vs baseline: 2.1673x; 1.2303x over previous
import functools

import jax
import jax.numpy as jnp
from jax import lax
from jax.experimental import pallas as pl
from jax.experimental.pallas import tpu as pltpu
from jax.experimental.pallas import tpu_sc as plsc

N = 4096
EDIM = 64
WFEAT = 64
LABELS = 2
SLOPE = 0.01
BLK = 512
NEG = -9e15
ICHUNK = 128


def _sc_gather_cols(embT, idx):
    info = plsc.get_sparse_core_info()
    nc, ns = info.num_cores, info.num_subcores
    nw = nc * ns
    d, v = embT.shape
    b = idx.shape[0]
    f_per_w = d // nw
    n_chunks = b // 16
    mesh = plsc.VectorSubcoreMesh(core_axis_name="c", subcore_axis_name="s")

    @functools.partial(
        pl.kernel,
        mesh=mesh,
        compiler_params=pltpu.CompilerParams(needs_layout_passes=False),
        out_type=jax.ShapeDtypeStruct((d * b,), jnp.float32),
        scratch_types=[
            pltpu.VMEM((v,), jnp.float32),
            pltpu.VMEM((b,), jnp.int32),
            pltpu.VMEM((b,), jnp.float32),
            pltpu.SemaphoreType.DMA,
            pltpu.SemaphoreType.DMA,
        ],
    )
    def k(table_hbm, idx_hbm, out_hbm, row_v, idx_v, res_v, sem, sem2):
        wid = lax.axis_index("s") * nc + lax.axis_index("c")
        pltpu.sync_copy(idx_hbm, idx_v)
        for f in range(f_per_w):
            c = wid * f_per_w + f
            pltpu.async_copy(table_hbm.at[c], row_v, sem).wait()

            def chunk_body(g, _):
                iv = idx_v[pl.ds(g * 16, 16)]
                res_v[pl.ds(g * 16, 16)] = plsc.load_gather(row_v, [iv])
                return 0

            lax.fori_loop(0, n_chunks, chunk_body, 0)
            pltpu.async_copy(res_v, out_hbm.at[pl.ds(c * b, b)], sem2).wait()

    return k(embT, idx)


def _gat_body(words_ref, w_ref, a1_ref, a2t_ref, wc_ref, bc_ref, adj_ref,
              att_ref, sent_ref, pool_ref, label_ref,
              wh_ref, f2t_ref, acc_ref):
    i = pl.program_id(0)
    nblk = pl.num_programs(0)

    @pl.when(i == 0)
    def _init():
        wh = lax.dot_general(
            words_ref[...], w_ref[...], (((0,), (0,)), ((), ())),
            preferred_element_type=jnp.float32)
        wh_ref[...] = wh
        f2t_ref[...] = lax.dot_general(
            a2t_ref[...], wh, (((1,), (1,)), ((), ())),
            preferred_element_type=jnp.float32)
        acc_ref[...] = jnp.zeros_like(acc_ref)

    wh_all = wh_ref[...]
    wh_blk = wh_ref[pl.ds(i * BLK, BLK), :]
    f1 = jnp.dot(wh_blk, a1_ref[...], preferred_element_type=jnp.float32)
    e = f1 + f2t_ref[...]
    e = jnp.where(e >= 0, e, SLOPE * e)
    e = jnp.where(adj_ref[...] > 0, e, NEG)
    m = jnp.max(e, axis=1, keepdims=True)
    p = jnp.exp(e - m)
    att = p / jnp.sum(p, axis=1, keepdims=True)
    att_ref[...] = att
    h = jnp.dot(att, wh_all, preferred_element_type=jnp.float32)
    sent = jnp.where(h > 0, h, jnp.exp(jnp.minimum(h, 0.0)) - 1.0)
    sent_ref[...] = sent
    acc_ref[...] += jnp.sum(sent, axis=0, keepdims=True)

    @pl.when(i == nblk - 1)
    def _fin():
        pool = acc_ref[...] * (1.0 / N)
        pool_ref[...] = pool
        logits = jnp.dot(pool, wc_ref[...],
                         preferred_element_type=jnp.float32) + bc_ref[...]
        mm = jnp.max(logits, axis=1, keepdims=True)
        pe = jnp.exp(logits - mm)
        label_ref[...] = pe / jnp.sum(pe, axis=1, keepdims=True)


def _gat(words, adj, W, a1, a2t, Wc, bc2):
    nblk = N // BLK
    out_shapes = (
        jax.ShapeDtypeStruct((N, N), jnp.float32),
        jax.ShapeDtypeStruct((N, WFEAT), jnp.float32),
        jax.ShapeDtypeStruct((1, WFEAT), jnp.float32),
        jax.ShapeDtypeStruct((1, LABELS), jnp.float32),
    )
    return pl.pallas_call(
        _gat_body,
        grid=(nblk,),
        in_specs=[
            pl.BlockSpec((EDIM, N), lambda i: (0, 0)),
            pl.BlockSpec((EDIM, WFEAT), lambda i: (0, 0)),
            pl.BlockSpec((WFEAT, 1), lambda i: (0, 0)),
            pl.BlockSpec((1, WFEAT), lambda i: (0, 0)),
            pl.BlockSpec((WFEAT, LABELS), lambda i: (0, 0)),
            pl.BlockSpec((1, LABELS), lambda i: (0, 0)),
            pl.BlockSpec((BLK, N), lambda i: (i, 0)),
        ],
        out_specs=(
            pl.BlockSpec((BLK, N), lambda i: (i, 0)),
            pl.BlockSpec((BLK, WFEAT), lambda i: (i, 0)),
            pl.BlockSpec((1, WFEAT), lambda i: (0, 0)),
            pl.BlockSpec((1, LABELS), lambda i: (0, 0)),
        ),
        out_shape=out_shapes,
        scratch_shapes=[
            pltpu.VMEM((N, WFEAT), jnp.float32),
            pltpu.VMEM((1, N), jnp.float32),
            pltpu.VMEM((1, WFEAT), jnp.float32),
        ],
    )(words, W, a1, a2t, Wc, bc2, adj)


def kernel(inSen, adj, emb, W, a, Wc, bc):
    idx = inSen.astype(jnp.int32)
    flat = _sc_gather_cols(emb.T, idx)
    wordsT = flat.reshape(EDIM, N)
    a1 = a[:WFEAT, :]
    a2t = a[WFEAT:, :].reshape(1, WFEAT)
    bc2 = bc.reshape(1, LABELS)
    attention, sentence, pool, label = _gat(wordsT, adj, W, a1, a2t, Wc, bc2)
    return (pool.reshape(WFEAT), attention, sentence, label.reshape(LABELS))


# speedup vs baseline: 2.2223x; 1.0254x over previous
import functools

import jax
import jax.numpy as jnp
from jax import lax
from jax.experimental import pallas as pl
from jax.experimental.pallas import tpu as pltpu
from jax.experimental.pallas import tpu_sc as plsc

N = 4096
EDIM = 64
WFEAT = 64
LABELS = 2
SLOPE = 0.01
BLK = 512
NEG = -9e15
ICHUNK = 128


def _sc_gather_cols(embT, idx):
    info = plsc.get_sparse_core_info()
    nc, ns = info.num_cores, info.num_subcores
    nw = nc * ns
    d, v = embT.shape
    b = idx.shape[0]
    f_per_w = d // nw
    n_chunks = b // 16
    mesh = plsc.VectorSubcoreMesh(core_axis_name="c", subcore_axis_name="s")

    @functools.partial(
        pl.kernel,
        mesh=mesh,
        compiler_params=pltpu.CompilerParams(needs_layout_passes=False),
        out_type=jax.ShapeDtypeStruct((d, b), jnp.float32),
        scratch_types=[
            pltpu.VMEM((v,), jnp.float32),
            pltpu.VMEM((b,), jnp.int32),
            pltpu.VMEM((b,), jnp.float32),
            pltpu.SemaphoreType.DMA,
            pltpu.SemaphoreType.DMA,
        ],
    )
    def k(table_hbm, idx_hbm, out_hbm, row_v, idx_v, res_v, sem, sem2):
        wid = lax.axis_index("s") * nc + lax.axis_index("c")
        pltpu.sync_copy(idx_hbm, idx_v)
        for f in range(f_per_w):
            c = wid * f_per_w + f
            pltpu.async_copy(table_hbm.at[c], row_v, sem).wait()

            def chunk_body(g, _):
                iv = idx_v[pl.ds(g * 16, 16)]
                res_v[pl.ds(g * 16, 16)] = plsc.load_gather(row_v, [iv])
                return 0

            lax.fori_loop(0, n_chunks, chunk_body, 0)
            pltpu.async_copy(res_v, out_hbm.at[c], sem2).wait()

    return k(embT, idx)


def _gat_body(words_ref, w_ref, a1_ref, a2t_ref, wc_ref, bc_ref, adj_ref,
              att_ref, sent_ref, pool_ref, label_ref,
              wh_ref, f2t_ref, acc_ref):
    i = pl.program_id(0)
    nblk = pl.num_programs(0)

    @pl.when(i == 0)
    def _init():
        wh = lax.dot_general(
            words_ref[...], w_ref[...], (((0,), (0,)), ((), ())),
            preferred_element_type=jnp.float32)
        wh_ref[...] = wh
        f2t_ref[...] = lax.dot_general(
            a2t_ref[...], wh, (((1,), (1,)), ((), ())),
            preferred_element_type=jnp.float32)
        acc_ref[...] = jnp.zeros_like(acc_ref)

    wh_all = wh_ref[...]
    wh_blk = wh_ref[pl.ds(i * BLK, BLK), :]
    f1 = jnp.dot(wh_blk, a1_ref[...], preferred_element_type=jnp.float32)
    e = f1 + f2t_ref[...]
    e = jnp.where(e >= 0, e, SLOPE * e)
    e = jnp.where(adj_ref[...] > 0, e, NEG)
    m = jnp.max(e, axis=1, keepdims=True)
    p = jnp.exp(e - m)
    att = p / jnp.sum(p, axis=1, keepdims=True)
    att_ref[...] = att
    h = jnp.dot(att, wh_all, preferred_element_type=jnp.float32)
    sent = jnp.where(h > 0, h, jnp.exp(jnp.minimum(h, 0.0)) - 1.0)
    sent_ref[...] = sent
    acc_ref[...] += jnp.sum(sent, axis=0, keepdims=True)

    @pl.when(i == nblk - 1)
    def _fin():
        pool = acc_ref[...] * (1.0 / N)
        pool_ref[...] = pool
        logits = jnp.dot(pool, wc_ref[...],
                         preferred_element_type=jnp.float32) + bc_ref[...]
        mm = jnp.max(logits, axis=1, keepdims=True)
        pe = jnp.exp(logits - mm)
        label_ref[...] = pe / jnp.sum(pe, axis=1, keepdims=True)


def _gat(words, adj, W, a1, a2t, Wc, bc2):
    nblk = N // BLK
    out_shapes = (
        jax.ShapeDtypeStruct((N, N), jnp.float32),
        jax.ShapeDtypeStruct((N, WFEAT), jnp.float32),
        jax.ShapeDtypeStruct((1, WFEAT), jnp.float32),
        jax.ShapeDtypeStruct((1, LABELS), jnp.float32),
    )
    return pl.pallas_call(
        _gat_body,
        grid=(nblk,),
        in_specs=[
            pl.BlockSpec((EDIM, N), lambda i: (0, 0)),
            pl.BlockSpec((EDIM, WFEAT), lambda i: (0, 0)),
            pl.BlockSpec((WFEAT, 1), lambda i: (0, 0)),
            pl.BlockSpec((1, WFEAT), lambda i: (0, 0)),
            pl.BlockSpec((WFEAT, LABELS), lambda i: (0, 0)),
            pl.BlockSpec((1, LABELS), lambda i: (0, 0)),
            pl.BlockSpec((BLK, N), lambda i: (i, 0)),
        ],
        out_specs=(
            pl.BlockSpec((BLK, N), lambda i: (i, 0)),
            pl.BlockSpec((BLK, WFEAT), lambda i: (i, 0)),
            pl.BlockSpec((1, WFEAT), lambda i: (0, 0)),
            pl.BlockSpec((1, LABELS), lambda i: (0, 0)),
        ),
        out_shape=out_shapes,
        scratch_shapes=[
            pltpu.VMEM((N, WFEAT), jnp.float32),
            pltpu.VMEM((1, N), jnp.float32),
            pltpu.VMEM((1, WFEAT), jnp.float32),
        ],
    )(words, W, a1, a2t, Wc, bc2, adj)


def kernel(inSen, adj, emb, W, a, Wc, bc):
    idx = inSen.astype(jnp.int32)
    wordsT = _sc_gather_cols(emb.T, idx)
    a1 = a[:WFEAT, :]
    a2t = a[WFEAT:, :].reshape(1, WFEAT)
    bc2 = bc.reshape(1, LABELS)
    attention, sentence, pool, label = _gat(wordsT, adj, W, a1, a2t, Wc, bc2)
    return (pool.reshape(WFEAT), attention, sentence, label.reshape(LABELS))
